# Initial kernel scaffold; baseline (speedup 1.0000x reference)
import jax
import jax.numpy as jnp
from jax.experimental import pallas as pl


def kernel(x, edge_index, edge_attr, conditions, batch, node_enc_w1, node_enc_b1, node_enc_w2, node_enc_b2, edge_enc_w1, edge_enc_b1, edge_enc_w2, edge_enc_b2, cond_enc_w1, cond_enc_b1, cond_enc_w2, cond_enc_b2, node_dec_w1, node_dec_b1, node_dec_w2, node_dec_b2, edge0_w1s, edge0_w1d, edge0_w1e, edge0_w1u, edge0_b1, edge0_w2, edge0_b2, node0_w1x, node0_w1a, node0_w1u, node0_b1, node0_w2, node0_b2, edge1_w1s, edge1_w1d, edge1_w1e, edge1_w1u, edge1_b1, edge1_w2, edge1_b2, node1_w1x, node1_w1a, node1_w1u, node1_b1, node1_w2, node1_b2):
    raise NotImplementedError("write your pallas kernel here")


# speedup vs baseline: 78.4842x; 78.4842x over previous
import functools

import jax
import jax.numpy as jnp
from jax.experimental import pallas as pl
from jax.experimental.pallas import tpu as pltpu

_F32 = jnp.float32


def _enc_body(x_ref, w1_ref, b1_ref, w2_ref, b2_ref, o_ref):
    h = jnp.dot(x_ref[...], w1_ref[...], preferred_element_type=_F32)
    h = jnp.maximum(h + b1_ref[...], 0.0)
    o_ref[...] = jnp.dot(h, w2_ref[...], preferred_element_type=_F32) + b2_ref[...]


def _encode(x, w1, b1, w2, b2, tile):
    rows, din = x.shape
    dh = w1.shape[1]
    dout = w2.shape[1]
    return pl.pallas_call(
        _enc_body,
        out_shape=jax.ShapeDtypeStruct((rows, dout), _F32),
        grid=(rows // tile,),
        in_specs=[
            pl.BlockSpec((tile, din), lambda i: (i, 0)),
            pl.BlockSpec((din, dh), lambda i: (0, 0)),
            pl.BlockSpec((1, dh), lambda i: (0, 0)),
            pl.BlockSpec((dh, dout), lambda i: (0, 0)),
            pl.BlockSpec((1, dout), lambda i: (0, 0)),
        ],
        out_specs=pl.BlockSpec((tile, dout), lambda i: (i, 0)),
        compiler_params=pltpu.CompilerParams(dimension_semantics=("arbitrary",)),
        name="encoder_mlp",
    )(x, w1, b1, w2, b2)


def _layer_body(km, bm, fs, ls, am,
                row_ref, src_ref, dst_ref, e_ref, ue_ref, x_ref, un_ref,
                w1s, w1d, w1e, w1u, b1, w2, b2,
                w1x, w1a, w1un, b1n, w2n, b2n,
                *rest, tn, fuse_dec):
    if fuse_dec:
        dw1, db1, dw2, db2, o_ref, sum_acc, cnt_acc = rest
    else:
        enew_ref, o_ref, sum_acc, cnt_acc = rest
    s = pl.program_id(0)

    h = jnp.dot(src_ref[...], w1s[...], preferred_element_type=_F32)
    h = h + jnp.dot(dst_ref[...], w1d[...], preferred_element_type=_F32)
    h = h + jnp.dot(e_ref[...], w1e[...], preferred_element_type=_F32)
    h = h + jnp.dot(ue_ref[...], w1u[...], preferred_element_type=_F32)
    h = jnp.maximum(h + b1[...], 0.0)
    msg = jnp.dot(h, w2[...], preferred_element_type=_F32) + b2[...]
    if not fuse_dec:
        enew_ref[...] = msg

    @pl.when(fs[s] == 1)
    def _init():
        sum_acc[...] = jnp.zeros_like(sum_acc)
        cnt_acc[...] = jnp.zeros_like(cnt_acc)

    te = row_ref.shape[1]
    nids = bm[s] * tn + jax.lax.broadcasted_iota(jnp.int32, (tn, te), 0)
    oh = (nids == row_ref[...]).astype(_F32) * am[s].astype(_F32)
    sum_acc[...] += jnp.dot(oh, msg, preferred_element_type=_F32)
    cnt_acc[...] += jnp.sum(oh, axis=1, keepdims=True)

    @pl.when(ls[s] == 1)
    def _finish():
        agg = sum_acc[...] / jnp.maximum(cnt_acc[...], 1.0)
        xb = x_ref[...]
        h2 = jnp.dot(xb, w1x[...], preferred_element_type=_F32)
        h2 = h2 + jnp.dot(agg, w1a[...], preferred_element_type=_F32)
        h2 = h2 + jnp.dot(un_ref[...], w1un[...], preferred_element_type=_F32)
        h2 = jnp.maximum(h2 + b1n[...], 0.0)
        xn = jnp.dot(h2, w2n[...], preferred_element_type=_F32) + b2n[...] + xb
        if fuse_dec:
            hd = jnp.dot(xn, dw1[...], preferred_element_type=_F32)
            hd = jnp.maximum(hd + db1[...], 0.0)
            o_ref[...] = jnp.dot(hd, dw2[...], preferred_element_type=_F32) + db2[...]
        else:
            o_ref[...] = xn


def _layer(row2d, src, dst, e_h, ue, x_h, un, ew, nw, dec, maps, *, tn, te):
    kmap, bmap, first, last, accm = maps
    e_tot, hp = src.shape
    n_tot = x_h.shape[0]
    steps = kmap.shape[0]
    fuse_dec = dec is not None

    def eix(s, km, bm, fs, ls, am):
        return (km[s], 0)

    def nix(s, km, bm, fs, ls, am):
        return (bm[s], 0)

    def rix(s, km, bm, fs, ls, am):
        return (0, km[s])

    cst = lambda s, *_: (0, 0)
    e_spec = pl.BlockSpec((te, hp), eix)
    n_spec = pl.BlockSpec((tn, hp), nix)
    w_spec = pl.BlockSpec((hp, hp), cst)
    b_spec = pl.BlockSpec((1, hp), cst)

    in_specs = [pl.BlockSpec((1, te), rix),
                e_spec, e_spec, e_spec, e_spec, n_spec, n_spec,
                w_spec, w_spec, w_spec, w_spec, b_spec, w_spec, b_spec,
                w_spec, w_spec, w_spec, b_spec, w_spec, b_spec]
    args = [row2d, src, dst, e_h, ue, x_h, un] + list(ew) + list(nw)
    if fuse_dec:
        in_specs += [w_spec, b_spec, w_spec, b_spec]
        args += list(dec)
        out_shape = jax.ShapeDtypeStruct((n_tot, hp), _F32)
        out_specs = n_spec
    else:
        out_shape = (jax.ShapeDtypeStruct((e_tot, hp), _F32),
                     jax.ShapeDtypeStruct((n_tot, hp), _F32))
        out_specs = (e_spec, n_spec)

    return pl.pallas_call(
        functools.partial(_layer_body, tn=tn, fuse_dec=fuse_dec),
        out_shape=out_shape,
        grid_spec=pltpu.PrefetchScalarGridSpec(
            num_scalar_prefetch=5,
            grid=(steps,),
            in_specs=in_specs,
            out_specs=out_specs,
            scratch_shapes=[pltpu.VMEM((tn, hp), _F32),
                            pltpu.VMEM((tn, 1), _F32)],
        ),
        compiler_params=pltpu.CompilerParams(
            dimension_semantics=("arbitrary",)),
        name="mp_layer",
    )(kmap, bmap, first, last, accm, *args)


def _visit_maps(row_s, n_tot, e_tot, tn, te):
    nb = n_tot // tn
    k = e_tot // te
    steps = k + nb
    bounds = jnp.searchsorted(row_s, jnp.arange(nb + 1, dtype=jnp.int32) * tn)
    lo = bounds[:-1]
    hi = bounds[1:]
    nonempty = hi > lo
    t0 = jnp.minimum(lo // te, k - 1)
    t1 = jnp.where(nonempty, jnp.maximum(hi - 1, 0) // te, t0)
    counts = jnp.where(nonempty, t1 - t0 + 1, 1)
    offs = jnp.concatenate([jnp.zeros((1,), jnp.int32),
                            jnp.cumsum(counts).astype(jnp.int32)])
    t_real = offs[-1]
    sidx = jnp.arange(steps, dtype=jnp.int32)
    bmap = jnp.clip(jnp.searchsorted(offs, sidx, side="right") - 1, 0, nb - 1)
    bmap = bmap.astype(jnp.int32)
    kmap = jnp.clip(t0[bmap] + (sidx - offs[bmap]), 0, k - 1).astype(jnp.int32)
    accm = (sidx < t_real).astype(jnp.int32)
    chg = (bmap[1:] != bmap[:-1]).astype(jnp.int32)
    one = jnp.ones((1,), jnp.int32)
    first = jnp.concatenate([one, chg])
    last = jnp.concatenate([chg, one])
    return kmap, bmap, first, last, accm


def _row_tile(n, pref):
    return pref if n % pref == 0 else 8


def kernel(x, edge_index, edge_attr, conditions, batch, node_enc_w1, node_enc_b1, node_enc_w2, node_enc_b2, edge_enc_w1, edge_enc_b1, edge_enc_w2, edge_enc_b2, cond_enc_w1, cond_enc_b1, cond_enc_w2, cond_enc_b2, node_dec_w1, node_dec_b1, node_dec_w2, node_dec_b2, edge0_w1s, edge0_w1d, edge0_w1e, edge0_w1u, edge0_b1, edge0_w2, edge0_b2, node0_w1x, node0_w1a, node0_w1u, node0_b1, node0_w2, node0_b2, edge1_w1s, edge1_w1d, edge1_w1e, edge1_w1u, edge1_b1, edge1_w2, edge1_b2, node1_w1x, node1_w1a, node1_w1u, node1_b1, node1_w2, node1_b2):
    n_tot = x.shape[0]
    e_tot = edge_attr.shape[0]
    hp = node_enc_w2.shape[1]
    node_out = 3

    tn = _row_tile(n_tot, 256)
    te = _row_tile(e_tot, 512)

    row = edge_index[0].astype(jnp.int32)
    col = edge_index[1].astype(jnp.int32)

    perm = jnp.argsort(row)
    row_s = row[perm]
    col_s = col[perm]
    ea_s = edge_attr[perm]
    ebatch_s = batch[row_s]

    maps = _visit_maps(row_s, n_tot, e_tot, tn, te)
    row2d = row_s.reshape(1, e_tot)

    x_h = _encode(x, node_enc_w1, node_enc_b1, node_enc_w2, node_enc_b2,
                  _row_tile(n_tot, 1024))
    e_h = _encode(ea_s, edge_enc_w1, edge_enc_b1, edge_enc_w2, edge_enc_b2,
                  _row_tile(e_tot, 1024))
    u_h = _encode(conditions, cond_enc_w1, cond_enc_b1, cond_enc_w2,
                  cond_enc_b2, _row_tile(conditions.shape[0], 128))

    u_e = u_h[ebatch_s]
    u_n = u_h[batch.astype(jnp.int32)]

    ew0 = (edge0_w1s, edge0_w1d, edge0_w1e, edge0_w1u, edge0_b1, edge0_w2,
           edge0_b2)
    nw0 = (node0_w1x, node0_w1a, node0_w1u, node0_b1, node0_w2, node0_b2)
    ew1 = (edge1_w1s, edge1_w1d, edge1_w1e, edge1_w1u, edge1_b1, edge1_w2,
           edge1_b2)
    nw1 = (node1_w1x, node1_w1a, node1_w1u, node1_b1, node1_w2, node1_b2)
    dec = (node_dec_w1, node_dec_b1, node_dec_w2, node_dec_b2)

    src = x_h[row_s]
    dst = x_h[col_s]
    e_h, x_h = _layer(row2d, src, dst, e_h, u_e, x_h, u_n, ew0, nw0, None,
                      maps, tn=tn, te=te)

    src = x_h[row_s]
    dst = x_h[col_s]
    out = _layer(row2d, src, dst, e_h, u_e, x_h, u_n, ew1, nw1, dec,
                 maps, tn=tn, te=te)
    return out[:, :node_out]
